# baseline (speedup 1.0000x reference)
import functools

import jax
import jax.numpy as jnp
from jax import lax
from jax.experimental import pallas as pl
from jax.experimental.pallas import tpu as pltpu
from jax.experimental.pallas import tpu_sc as plsc

N_DET = 16
N_SO = 512
N_E = 32
BATCH = 4096
LANES = 128

_info = plsc.get_sparse_core_info()
NC, NS, L = _info.num_cores, _info.num_subcores, _info.num_lanes
NW = NC * NS

BH = BATCH // 2
NBT = BH // LANES
JT = 8
N_JT = N_E // JT
N_STEPS = N_E * N_JT


def _spo_body(occ_hbm, tab_hbm, out_hbm,
              occ_v, tab_v, buf_a, buf_b, buf_c, buf_d,
              wsem_a, wsem_b, wsem_c, wsem_d):
    wid = lax.axis_index("s") * NC + lax.axis_index("c")
    d = wid // 2
    h = wid % 2
    b0 = h * BH

    pltpu.sync_copy(tab_hbm.at[d], tab_v)
    pltpu.sync_copy(occ_hbm.at[:, pl.ds(b0, BH)], occ_v)

    def produce(e, jt, k, buf):
        jbase = jt * JT
        o_base = k * (BH // 2)

        @plsc.parallel_loop(0, BH // (2 * L), unroll=4)
        def bg_body(g):
            o = occ_v[e, pl.ds(o_base + g * L, L)]
            bt = g // 8
            lo = (g % 8) * L
            for j in range(JT):
                idx = o + (jbase + j) * N_SO
                buf[bt, j, pl.ds(lo, L)] = plsc.load_gather(tab_v, [idx])

    def fire_wb(e, jt, k, buf, sem):
        pltpu.async_copy(
            buf,
            out_hbm.at[d, e, jt, pl.ds(h * NBT + k * (NBT // 2), NBT // 2)],
            sem)

    def drain_wb(buf, sem):
        pltpu.make_async_copy(buf, out_hbm.at[d, 0, 0, pl.ds(0, NBT // 2)],
                              sem).wait()

    bufs = (buf_a, buf_b, buf_c, buf_d)
    sems = (wsem_a, wsem_b, wsem_c, wsem_d)

    def quad(q, carry):
        for i in range(4):
            t = 4 * q + i
            e = t // (2 * N_JT)
            jt = (t % (2 * N_JT)) // 2
            k = t % 2

            @pl.when(q > 0)
            def _(buf=bufs[i], sem=sems[i]):
                drain_wb(buf, sem)
            produce(e, jt, k, bufs[i])
            fire_wb(e, jt, k, bufs[i], sems[i])
        return carry

    lax.fori_loop(0, 2 * N_STEPS // 4, quad, 0)
    for i in range(4):
        drain_wb(bufs[i], sems[i])


@functools.partial(jax.jit, static_argnames=())
def kernel(occ_so, phi_ref):
    occ_t = occ_so.astype(jnp.int32).T
    tab = phi_ref.transpose(0, 2, 1).reshape(N_DET, N_E * N_SO)

    mesh = plsc.VectorSubcoreMesh(core_axis_name="c", subcore_axis_name="s")
    out6 = pl.kernel(
        _spo_body,
        mesh=mesh,
        compiler_params=pltpu.CompilerParams(needs_layout_passes=False),
        out_type=jax.ShapeDtypeStruct(
            (N_DET, N_E, N_JT, BATCH // LANES, JT, LANES), jnp.float32),
        scratch_types=[
            pltpu.VMEM((N_E, BH), jnp.int32),
            pltpu.VMEM((N_SO * N_E,), jnp.float32),
            pltpu.VMEM((NBT // 2, JT, LANES), jnp.float32),
            pltpu.VMEM((NBT // 2, JT, LANES), jnp.float32),
            pltpu.VMEM((NBT // 2, JT, LANES), jnp.float32),
            pltpu.VMEM((NBT // 2, JT, LANES), jnp.float32),
            pltpu.SemaphoreType.DMA,
            pltpu.SemaphoreType.DMA,
            pltpu.SemaphoreType.DMA,
            pltpu.SemaphoreType.DMA,
        ],
    )(occ_t, tab)
    out = out6.transpose(3, 5, 0, 1, 2, 4).reshape(BATCH, N_DET, N_E, N_E)
    return out


# speedup vs baseline: 1.0116x; 1.0116x over previous
import functools

import jax
import jax.numpy as jnp
from jax import lax
from jax.experimental import pallas as pl
from jax.experimental.pallas import tpu as pltpu
from jax.experimental.pallas import tpu_sc as plsc

N_DET = 16
N_SO = 512
N_E = 32
BATCH = 4096
LANES = 128

_info = plsc.get_sparse_core_info()
NC, NS, L = _info.num_cores, _info.num_subcores, _info.num_lanes
NW = NC * NS

BH = BATCH // 2
NBT = BH // LANES
JT = 8
N_JT = N_E // JT
N_STEPS = N_E * N_JT


def _spo_body(occ_hbm, tab_hbm, out_hbm,
              occ_v, tab_v, buf_a, buf_b, wsem_a, wsem_b):
    wid = lax.axis_index("s") * NC + lax.axis_index("c")
    d = wid // 2
    h = wid % 2
    b0 = h * BH

    pltpu.sync_copy(tab_hbm.at[d], tab_v)
    pltpu.sync_copy(occ_hbm.at[:, pl.ds(b0, BH)], occ_v)

    def produce(e, jt, buf):
        jbase = jt * JT

        @plsc.parallel_loop(0, BH // L, unroll=4)
        def bg_body(g):
            o = occ_v[e, pl.ds(g * L, L)]
            bt = g // 8
            lo = (g % 8) * L
            for j in range(JT):
                idx = o + (jbase + j) * N_SO
                buf[bt, j, pl.ds(lo, L)] = plsc.load_gather(tab_v, [idx])

    def fire_wb(e, jt, buf, sem):
        pltpu.async_copy(buf, out_hbm.at[d, e, jt, pl.ds(h * NBT, NBT)], sem)

    def drain_wb(buf, sem):
        pltpu.make_async_copy(buf, out_hbm.at[d, 0, 0, pl.ds(0, NBT)],
                              sem).wait()

    def pair(p, carry):
        ta = 2 * p
        tb = ta + 1
        ea, jta = ta // N_JT, ta % N_JT
        eb, jtb = tb // N_JT, tb % N_JT

        @pl.when(p > 0)
        def _():
            drain_wb(buf_a, wsem_a)
        produce(ea, jta, buf_a)
        fire_wb(ea, jta, buf_a, wsem_a)

        @pl.when(p > 0)
        def _():
            drain_wb(buf_b, wsem_b)
        produce(eb, jtb, buf_b)
        fire_wb(eb, jtb, buf_b, wsem_b)
        return carry

    lax.fori_loop(0, N_STEPS // 2, pair, 0)
    drain_wb(buf_a, wsem_a)
    drain_wb(buf_b, wsem_b)


@functools.partial(jax.jit, static_argnames=())
def kernel(occ_so, phi_ref):
    occ_t = occ_so.astype(jnp.int32).T
    tab = phi_ref.transpose(0, 2, 1).reshape(N_DET, N_E * N_SO)

    mesh = plsc.VectorSubcoreMesh(core_axis_name="c", subcore_axis_name="s")
    out6 = pl.kernel(
        _spo_body,
        mesh=mesh,
        compiler_params=pltpu.CompilerParams(needs_layout_passes=False),
        out_type=jax.ShapeDtypeStruct(
            (N_DET, N_E, N_JT, BATCH // LANES, JT, LANES), jnp.float32),
        scratch_types=[
            pltpu.VMEM((N_E, BH), jnp.int32),
            pltpu.VMEM((N_SO * N_E,), jnp.float32),
            pltpu.VMEM((NBT, JT, LANES), jnp.float32),
            pltpu.VMEM((NBT, JT, LANES), jnp.float32),
            pltpu.SemaphoreType.DMA,
            pltpu.SemaphoreType.DMA,
        ],
    )(occ_t, tab)
    out = out6.transpose(3, 5, 0, 1, 2, 4).reshape(BATCH, N_DET, N_E, N_E)
    return out
